# baseline (speedup 1.0000x reference)
import functools

import jax
import jax.numpy as jnp
from jax import lax
from jax.experimental import pallas as pl
from jax.experimental.pallas import tpu as pltpu
from jax.experimental.pallas import tpu_sc as plsc

DATA_DIM = 28
ROWW = 32
N_NODES = 524288
N_CORNERS = 600000
N_QUERIES = 262144

NC = 2
NS = 16
L = 16
NW = NC * NS
QPW = N_QUERIES // NW
CHUNK = 128
NCHUNK = QPW // CHUNK


def _body(indices_hbm, nids_hbm, data_hbm, weights_hbm, out_hbm,
          idx_v, cid_v, cflat_v, rows_v, w_v, out_v, sem_n, sem_d):
    wid = lax.axis_index("s") * NC + lax.axis_index("c")
    base = wid * QPW

    iota = lax.iota(jnp.int32, L)
    hi = iota >> 3
    lo = iota & 7

    def chunk_body(g, _):
        qbase = pl.multiple_of(base + g * CHUNK, CHUNK)
        pltpu.sync_copy(indices_hbm.at[pl.ds(qbase, CHUNK)], idx_v)
        pltpu.async_copy(nids_hbm.at[idx_v], cid_v, sem_n).wait()
        for t in range(CHUNK // 2):
            idx_c = 2 * t + hi
            cvec = plsc.load_gather(cid_v, [idx_c, lo])
            cflat_v[t // 8, pl.ds((t % 8) * L, L)] = cvec
        copies = [
            pltpu.async_copy(data_hbm.at[cflat_v.at[k]], rows_v.at[k], sem_d)
            for k in range(8)
        ]
        for c in copies:
            c.wait()
        pltpu.sync_copy(weights_hbm.at[pl.ds(qbase, CHUNK), :], w_v)

        def q_body(c2, _):
            k = c2 // 8
            m = (c2 % 8) * L
            wv = plsc.load_gather(w_v, [2 * c2 + hi, lo])
            for h, c in ((0, 2 * c2), (8, 2 * c2 + 1)):
                w0 = wv[h]
                acc_lo = w0 * rows_v[k, m + h, pl.ds(0, L)]
                acc_hi = w0 * rows_v[k, m + h, pl.ds(DATA_DIM - L, L)]
                for j in range(1, 8):
                    wj = wv[h + j]
                    acc_lo = acc_lo + wj * rows_v[k, m + h + j, pl.ds(0, L)]
                    acc_hi = acc_hi + wj * rows_v[k, m + h + j, pl.ds(DATA_DIM - L, L)]
                out_v[c, pl.ds(0, L)] = acc_lo
                out_v[c, pl.ds(DATA_DIM - L, L)] = acc_hi
            return 0

        lax.fori_loop(0, CHUNK // 2, q_body, 0)
        pltpu.sync_copy(out_v, out_hbm.at[pl.ds(qbase, CHUNK), :])
        return 0

    lax.fori_loop(0, NCHUNK, chunk_body, 0)


@jax.jit
def kernel(indices, nids, data, weights):
    mesh = plsc.VectorSubcoreMesh(core_axis_name="c", subcore_axis_name="s")
    run = functools.partial(
        pl.kernel,
        mesh=mesh,
        out_type=jax.ShapeDtypeStruct((N_QUERIES, DATA_DIM), jnp.float32),
        compiler_params=pltpu.CompilerParams(
            needs_layout_passes=False, use_tc_tiling_on_sc=False),
        scratch_types=[
            pltpu.VMEM((CHUNK,), jnp.int32),
            pltpu.VMEM((CHUNK, 8), jnp.int32),
            pltpu.VMEM((8, CHUNK), jnp.int32),
            pltpu.VMEM((8, CHUNK, ROWW), jnp.float32),
            pltpu.VMEM((CHUNK, 8), jnp.float32),
            pltpu.VMEM((CHUNK, DATA_DIM), jnp.float32),
            pltpu.SemaphoreType.DMA,
            pltpu.SemaphoreType.DMA,
        ],
    )(_body)
    data_p = jnp.concatenate(
        [data, jnp.zeros((N_CORNERS, ROWW - DATA_DIM), jnp.float32)], axis=1)
    return run(indices, nids, data_p, weights)
